# Initial kernel scaffold; baseline (speedup 1.0000x reference)
import jax
import jax.numpy as jnp
from jax.experimental import pallas as pl


def kernel(pred, target):
    raise NotImplementedError("write your pallas kernel here")


# speedup vs baseline: 3.7358x; 3.7358x over previous
import math

import jax
import jax.numpy as jnp
from jax.experimental import pallas as pl
from jax.experimental.pallas import tpu as pltpu

_T = 10.0
_NR = 32
_LANES = 128
_RPB = 448

_K1 = 1.0 + math.exp(-_T)
_K2 = math.exp(-_T / 2.0)


def _loss_kernel(p_ref, t_ref, o_ref):
    i = pl.program_id(0)
    p = p_ref[...]
    t = t_ref[...]
    base = p * _T - (_T / 2.0)
    tidx = jnp.ceil(jnp.clip(t, 1.0, float(_NR))) - 1.0
    scale = jnp.exp2(tidx)
    den = jnp.zeros_like(p)
    num = jnp.zeros_like(p)
    for r in range(_NR):
        c = base - (_T * r)
        n = jnp.exp(-jnp.abs(c))
        u = n / (_K2 + n * (_K1 + _K2 * n))
        den = den + u
        num = num + jnp.where(tidx <= float(r), u * (2.0 ** (-r)), 0.0)
    expected = (num * scale) / (den + 1e-8)
    block_sum = jnp.sum(expected).reshape(1, 1)

    @pl.when(i == 0)
    def _():
        o_ref[...] = jnp.zeros_like(o_ref)

    o_ref[...] += block_sum


def kernel(pred, target):
    nb = pred.shape[0]
    be = _RPB * _LANES
    g = -(-nb // be)
    padded = g * be
    pp = jnp.pad(pred, (0, padded - nb), constant_values=-1e6)
    tt = jnp.pad(target, (0, padded - nb), constant_values=1.0)
    pp = pp.reshape(g * _RPB, _LANES)
    tt = tt.reshape(g * _RPB, _LANES)
    out = pl.pallas_call(
        _loss_kernel,
        out_shape=jax.ShapeDtypeStruct((1, 1), jnp.float32),
        grid=(g,),
        in_specs=[pl.BlockSpec((_RPB, _LANES), lambda i: (i, 0)),
                  pl.BlockSpec((_RPB, _LANES), lambda i: (i, 0))],
        out_specs=pl.BlockSpec((1, 1), lambda i: (0, 0)),
        compiler_params=pltpu.CompilerParams(
            dimension_semantics=("arbitrary",)),
        name="scband_loss",
    )(pp, tt)
    return 1.0 - out[0, 0] / nb
